# baseline (speedup 1.0000x reference)
import functools

import jax
import jax.numpy as jnp
from jax import lax
from jax.experimental import pallas as pl
from jax.experimental.pallas import tpu as pltpu
from jax.experimental.pallas import tpu_sc as plsc

_N = 10000
_D = 256
_E = 160000
_HP = 10240
_NC = 2
_NS = 16
_NW = _NC * _NS
_CHUNK = 128
_R = _E // _CHUNK
_K = 40
_KL = _R - 31 * _K
_B = 10
_SLICE = _HP // _NS


def _deg_body(ei_hbm, out_hbm, idx_v, ones_v, zero_v, hist_s, scat_sem):
    cid = lax.axis_index("c")
    sid = lax.axis_index("s")
    wid = sid * _NC + cid

    for i in range(_CHUNK // 16):
        ones_v[pl.ds(i * 16, 16)] = jnp.ones((16,), jnp.int32)
    for i in range(_SLICE // 16):
        zero_v[pl.ds(i * 16, 16)] = jnp.zeros((16,), jnp.int32)

    pltpu.sync_copy(zero_v, hist_s.at[pl.ds(sid * _SLICE, _SLICE)])

    def stage(j, carry):
        pltpu.sync_copy(
            ei_hbm.at[pl.ds(0, _NC), pl.ds((wid * _K + j) * _CHUNK, _CHUNK)],
            idx_v.at[pl.ds(2 * j, 2)],
        )
        return carry

    nrows = jnp.where(wid < _NW - 1, _K, _KL)
    lax.fori_loop(0, nrows, stage, 0)

    plsc.subcore_barrier()

    def batch(b, carry):
        descs = [
            pltpu.async_copy(ones_v, hist_s.at[idx_v.at[2 * (b * _B + u) + 1]],
                             scat_sem, add=True)
            for u in range(_B)
        ]
        for d in descs:
            d.wait()
        return carry

    nbatch = jnp.where(wid < _NW - 1, _K // _B, _KL // _B)
    lax.fori_loop(0, nbatch, batch, 0)

    plsc.subcore_barrier()

    pltpu.sync_copy(
        hist_s.at[pl.ds(sid * _SLICE, _SLICE)],
        out_hbm.at[cid, pl.ds(sid * _SLICE, _SLICE)],
    )


_deg_call = functools.partial(
    pl.kernel,
    out_type=jax.ShapeDtypeStruct((_NC, _HP), jnp.int32),
    mesh=plsc.VectorSubcoreMesh(
        core_axis_name="c", subcore_axis_name="s",
        num_cores=_NC, num_subcores=_NS,
    ),
    scratch_types=[
        pltpu.VMEM((2 * _K, _CHUNK), jnp.int32),
        pltpu.VMEM((_CHUNK,), jnp.int32),
        pltpu.VMEM((_SLICE,), jnp.int32),
        pltpu.VMEM_SHARED((_HP,), jnp.int32),
        pltpu.SemaphoreType.DMA,
    ],
    compiler_params=pltpu.CompilerParams(use_tc_tiling_on_sc=True),
)(_deg_body)

_BN = 1024


def _pow_body(h_ref, hist_ref, o_ref):
    h = h_ref[...]
    hl = hist_ref[...]
    deg_l = (hl[0] + hl[1]).reshape(1, _BN)
    deg = lax.transpose(deg_l, (1, 0))
    e = jnp.maximum(deg, 1)
    ef = e.astype(jnp.float32)
    r = jnp.exp2(ef * jnp.log2(jnp.abs(h)))
    neg = (h < 0.0) & ((e & 1) == 1)
    r = jnp.where(neg, -r, r)
    o_ref[...] = jnp.where(deg == 0, h, r)


_pow_call = pl.pallas_call(
    _pow_body,
    grid=(pl.cdiv(_N, _BN),),
    in_specs=[
        pl.BlockSpec((_BN, _D), lambda i: (i, 0)),
        pl.BlockSpec((_NC, _BN), lambda i: (0, i)),
    ],
    out_specs=pl.BlockSpec((_BN, _D), lambda i: (i, 0)),
    out_shape=jax.ShapeDtypeStruct((_N, _D), jnp.float32),
)


def kernel(h, edge_index):
    hist = _deg_call(edge_index)
    return _pow_call(h, hist)


# speedup vs baseline: 1.4327x; 1.4327x over previous
import functools

import jax
import jax.numpy as jnp
from jax import lax
from jax.experimental import pallas as pl
from jax.experimental.pallas import tpu as pltpu
from jax.experimental.pallas import tpu_sc as plsc

_N = 10000
_D = 256
_E = 160000
_HP = 10240
_NC = 2
_NS = 16
_NW = _NC * _NS
_CHUNK = 128
_R = _E // _CHUNK
_K = 40
_KL = _R - 31 * _K
_B = 10
_SLICE = _HP // _NS


def _deg_body(ei_hbm, out_hbm, idx_v, ones_v, zero_v, hist_s, scat_sem):
    cid = lax.axis_index("c")
    sid = lax.axis_index("s")
    wid = sid * _NC + cid

    for i in range(_CHUNK // 16):
        ones_v[pl.ds(i * 16, 16)] = jnp.ones((16,), jnp.int32)
    for i in range(_SLICE // 16):
        zero_v[pl.ds(i * 16, 16)] = jnp.zeros((16,), jnp.int32)

    pltpu.sync_copy(zero_v, hist_s.at[pl.ds(sid * _SLICE, _SLICE)])

    def stage_batch(b, carry):
        descs = [
            pltpu.async_copy(
                ei_hbm.at[pl.ds(0, _NC),
                          pl.ds((wid * _K + b * _B + u) * _CHUNK, _CHUNK)],
                idx_v.at[pl.ds(2 * (b * _B + u), 2)],
                scat_sem,
            )
            for u in range(_B)
        ]
        for d in descs:
            d.wait()
        return carry

    nbatch0 = jnp.where(wid < _NW - 1, _K // _B, _KL // _B)
    lax.fori_loop(0, nbatch0, stage_batch, 0)

    plsc.subcore_barrier()

    def batch(b, carry):
        descs = [
            pltpu.async_copy(ones_v, hist_s.at[idx_v.at[2 * (b * _B + u) + 1]],
                             scat_sem, add=True)
            for u in range(_B)
        ]
        for d in descs:
            d.wait()
        return carry

    nbatch = jnp.where(wid < _NW - 1, _K // _B, _KL // _B)
    lax.fori_loop(0, nbatch, batch, 0)

    plsc.subcore_barrier()

    pltpu.sync_copy(
        hist_s.at[pl.ds(sid * _SLICE, _SLICE)],
        out_hbm.at[cid, pl.ds(sid * _SLICE, _SLICE)],
    )


_deg_call = functools.partial(
    pl.kernel,
    out_type=jax.ShapeDtypeStruct((_NC, _HP), jnp.int32),
    mesh=plsc.VectorSubcoreMesh(
        core_axis_name="c", subcore_axis_name="s",
        num_cores=_NC, num_subcores=_NS,
    ),
    scratch_types=[
        pltpu.VMEM((2 * _K, _CHUNK), jnp.int32),
        pltpu.VMEM((_CHUNK,), jnp.int32),
        pltpu.VMEM((_SLICE,), jnp.int32),
        pltpu.VMEM_SHARED((_HP,), jnp.int32),
        pltpu.SemaphoreType.DMA,
    ],
    compiler_params=pltpu.CompilerParams(use_tc_tiling_on_sc=True),
)(_deg_body)

_BN = 1024


def _pow_body(h_ref, hist_ref, o_ref):
    h = h_ref[...]
    hl = hist_ref[...]
    deg_l = (hl[0] + hl[1]).reshape(1, _BN)
    deg = lax.transpose(deg_l, (1, 0))
    e = jnp.maximum(deg, 1)
    ef = e.astype(jnp.float32)
    r = jnp.exp2(ef * jnp.log2(jnp.abs(h)))
    neg = (h < 0.0) & ((e & 1) == 1)
    r = jnp.where(neg, -r, r)
    o_ref[...] = jnp.where(deg == 0, h, r)


_pow_call = pl.pallas_call(
    _pow_body,
    grid=(pl.cdiv(_N, _BN),),
    in_specs=[
        pl.BlockSpec((_BN, _D), lambda i: (i, 0)),
        pl.BlockSpec((_NC, _BN), lambda i: (0, i)),
    ],
    out_specs=pl.BlockSpec((_BN, _D), lambda i: (i, 0)),
    out_shape=jax.ShapeDtypeStruct((_N, _D), jnp.float32),
)


def kernel(h, edge_index):
    hist = _deg_call(edge_index)
    return _pow_call(h, hist)


# speedup vs baseline: 1.5040x; 1.0498x over previous
import functools

import jax
import jax.numpy as jnp
from jax import lax
from jax.experimental import pallas as pl
from jax.experimental.pallas import tpu as pltpu
from jax.experimental.pallas import tpu_sc as plsc

_N = 10000
_D = 256
_E = 160000
_HP = 10240
_NC = 2
_NS = 16
_NW = _NC * _NS
_CHUNK = 128
_R = _E // _CHUNK
_K = 40
_KL = _R - 31 * _K
_B = 10
_SLICE = _HP // _NS


def _deg_body(ei_hbm, out_hbm, idx_v, ones_v, zero_v, hist_s, scat_sem):
    cid = lax.axis_index("c")
    sid = lax.axis_index("s")
    wid = sid * _NC + cid

    for i in range(_CHUNK // 16):
        ones_v[pl.ds(i * 16, 16)] = jnp.ones((16,), jnp.int32)
    for i in range(_SLICE // 16):
        zero_v[pl.ds(i * 16, 16)] = jnp.zeros((16,), jnp.int32)

    pltpu.sync_copy(zero_v, hist_s.at[pl.ds(sid * _SLICE, _SLICE)])

    def stage_batch(b, carry):
        descs = [
            pltpu.async_copy(
                ei_hbm.at[pl.ds(0, _NC),
                          pl.ds((wid * _K + b * _B + u) * _CHUNK, _CHUNK)],
                idx_v.at[pl.ds(2 * (b * _B + u), 2)],
                scat_sem,
            )
            for u in range(_B)
        ]
        for d in descs:
            d.wait()
        return carry

    nbatch0 = jnp.where(wid < _NW - 1, _K // _B, _KL // _B)
    lax.fori_loop(0, nbatch0, stage_batch, 0)

    plsc.subcore_barrier()

    def batch(b, carry):
        descs = [
            pltpu.async_copy(ones_v, hist_s.at[idx_v.at[2 * (b * _B + u) + 1]],
                             scat_sem, add=True)
            for u in range(_B)
        ]
        for d in descs:
            d.wait()
        return carry

    nbatch = jnp.where(wid < _NW - 1, _K // _B, _KL // _B)
    lax.fori_loop(0, nbatch, batch, 0)

    plsc.subcore_barrier()

    pltpu.sync_copy(
        hist_s.at[pl.ds(sid * _SLICE, _SLICE)],
        out_hbm.at[cid, pl.ds(sid * _SLICE, _SLICE)],
    )


_deg_call = functools.partial(
    pl.kernel,
    out_type=jax.ShapeDtypeStruct((_NC, _HP), jnp.int32),
    mesh=plsc.VectorSubcoreMesh(
        core_axis_name="c", subcore_axis_name="s",
        num_cores=_NC, num_subcores=_NS,
    ),
    scratch_types=[
        pltpu.VMEM((2 * _K, _CHUNK), jnp.int32),
        pltpu.VMEM((_CHUNK,), jnp.int32),
        pltpu.VMEM((_SLICE,), jnp.int32),
        pltpu.VMEM_SHARED((_HP,), jnp.int32),
        pltpu.SemaphoreType.DMA,
    ],
    compiler_params=pltpu.CompilerParams(use_tc_tiling_on_sc=True),
)(_deg_body)

_BN = 2048


def _pow_body(h_ref, hist_ref, o_ref):
    h = h_ref[...]
    hl = hist_ref[...]
    deg_l = (hl[0] + hl[1]).reshape(1, _BN)
    deg = lax.transpose(deg_l, (1, 0))
    e = jnp.maximum(deg, 1)
    ef = e.astype(jnp.float32)
    r = jnp.exp2(ef * jnp.log2(jnp.abs(h)))
    neg = (h < 0.0) & ((e & 1) == 1)
    r = jnp.where(neg, -r, r)
    o_ref[...] = jnp.where(deg == 0, h, r)


_pow_call = pl.pallas_call(
    _pow_body,
    grid=(pl.cdiv(_N, _BN),),
    in_specs=[
        pl.BlockSpec((_BN, _D), lambda i: (i, 0)),
        pl.BlockSpec((_NC, _BN), lambda i: (0, i)),
    ],
    out_specs=pl.BlockSpec((_BN, _D), lambda i: (i, 0)),
    out_shape=jax.ShapeDtypeStruct((_N, _D), jnp.float32),
)


def kernel(h, edge_index):
    hist = _deg_call(edge_index)
    return _pow_call(h, hist)
